# baseline (speedup 1.0000x reference)
import functools
import jax
import jax.numpy as jnp
from jax.experimental import pallas as pl
from jax.experimental.pallas import tpu as pltpu

N_LAST = 50000
N_PARTIAL = 25000
N_POINTS = 100000
N_SCALE = 12500
C = 128
KEYSPACE = 524288
TOTAL = float(2 * 128 * 128 * 16)
EPS = 1e-5


def _row_grid(n, r):
    assert n % r == 0
    return n // r


def _bspec(r, c, const=False):
    if const:
        return pl.BlockSpec((r, c), lambda g: (0, 0))
    return pl.BlockSpec((r, c), lambda g: (g, 0))


def _mm_body(x_ref, w_ref, b_ref, y_ref, st_ref, acc):
    g = pl.program_id(0)
    y = jnp.dot(x_ref[...], w_ref[...], preferred_element_type=jnp.float32) + b_ref[...]
    y_ref[...] = y

    @pl.when(g == 0)
    def _():
        acc[...] = jnp.zeros_like(acc)

    acc[...] += jnp.stack([jnp.sum(y, axis=0), jnp.sum(y * y, axis=0)])
    st_ref[...] = acc[...]


def _mm(x, w, b, r):
    n, cin = x.shape
    cout = w.shape[1]
    return pl.pallas_call(
        _mm_body,
        grid=(_row_grid(n, r),),
        in_specs=[_bspec(r, cin), _bspec(cin, cout, True), _bspec(1, cout, True)],
        out_specs=[_bspec(r, cout), _bspec(2, cout, True)],
        out_shape=[jax.ShapeDtypeStruct((n, cout), jnp.float32),
                   jax.ShapeDtypeStruct((2, cout), jnp.float32)],
        scratch_shapes=[pltpu.VMEM((2, cout), jnp.float32)],
        compiler_params=pltpu.CompilerParams(dimension_semantics=("arbitrary",)),
    )(x, w, b.reshape(1, -1))


def _norm(y, st, nf):
    m = st[0:1, :] * (1.0 / nf)
    v = st[1:2, :] * (1.0 / nf) - m * m
    return (y - m) * jax.lax.rsqrt(v + EPS)


def _nrm_body(nf, y_ref, st_in, w_ref, b_ref, y2_ref, st_ref, acc):
    g = pl.program_id(0)
    z = jax.nn.relu(_norm(y_ref[...], st_in[...], nf))
    y2 = jnp.dot(z, w_ref[...], preferred_element_type=jnp.float32) + b_ref[...]
    y2_ref[...] = y2

    @pl.when(g == 0)
    def _():
        acc[...] = jnp.zeros_like(acc)

    acc[...] += jnp.stack([jnp.sum(y2, axis=0), jnp.sum(y2 * y2, axis=0)])
    st_ref[...] = acc[...]


def _nrm(y, st, w, b, r):
    n, cin = y.shape
    cout = w.shape[1]
    return pl.pallas_call(
        functools.partial(_nrm_body, float(n)),
        grid=(_row_grid(n, r),),
        in_specs=[_bspec(r, cin), _bspec(2, cin, True), _bspec(cin, cout, True),
                  _bspec(1, cout, True)],
        out_specs=[_bspec(r, cout), _bspec(2, cout, True)],
        out_shape=[jax.ShapeDtypeStruct((n, cout), jnp.float32),
                   jax.ShapeDtypeStruct((2, cout), jnp.float32)],
        scratch_shapes=[pltpu.VMEM((2, cout), jnp.float32)],
        compiler_params=pltpu.CompilerParams(dimension_semantics=("arbitrary",)),
    )(y, st, w, b.reshape(1, -1))


def _nrmres_body(nf, y_ref, st_in, xres_ref, w_ref, b_ref, x2_ref, y2_ref, st_ref, acc):
    g = pl.program_id(0)
    x2 = jax.nn.relu(_norm(y_ref[...], st_in[...], nf) + xres_ref[...])
    x2_ref[...] = x2
    y2 = jnp.dot(x2, w_ref[...], preferred_element_type=jnp.float32) + b_ref[...]
    y2_ref[...] = y2

    @pl.when(g == 0)
    def _():
        acc[...] = jnp.zeros_like(acc)

    acc[...] += jnp.stack([jnp.sum(y2, axis=0), jnp.sum(y2 * y2, axis=0)])
    st_ref[...] = acc[...]


def _nrmres(y, st, xres, w, b, r):
    n, cin = y.shape
    cout = w.shape[1]
    return pl.pallas_call(
        functools.partial(_nrmres_body, float(n)),
        grid=(_row_grid(n, r),),
        in_specs=[_bspec(r, cin), _bspec(2, cin, True), _bspec(r, cin),
                  _bspec(cin, cout, True), _bspec(1, cout, True)],
        out_specs=[_bspec(r, cin), _bspec(r, cout), _bspec(2, cout, True)],
        out_shape=[jax.ShapeDtypeStruct((n, cin), jnp.float32),
                   jax.ShapeDtypeStruct((n, cout), jnp.float32),
                   jax.ShapeDtypeStruct((2, cout), jnp.float32)],
        scratch_shapes=[pltpu.VMEM((2, cout), jnp.float32)],
        compiler_params=pltpu.CompilerParams(dimension_semantics=("arbitrary",)),
    )(y, st, xres, w, b.reshape(1, -1))


def _lrelu(x):
    return jnp.where(x > 0, x, 0.1 * x)


def _tailf_body(nf, y4_ref, st_in, x2_ref, x_ref, wpi_ref, bpi_ref, wtop_ref,
                b1_ref, feat_ref, a_ref):
    v = jax.nn.relu(_norm(y4_ref[...], st_in[...], nf) + x2_ref[...])
    feat = x_ref[...] + v
    feat_ref[...] = feat
    t = _lrelu(jnp.dot(feat, wpi_ref[...], preferred_element_type=jnp.float32)
               + bpi_ref[...])
    a_ref[...] = jnp.dot(t, wtop_ref[...], preferred_element_type=jnp.float32) + b1_ref[...]


def _tailf(y4, st, x2, x, wpi, bpi, wtop, b1, r):
    n = y4.shape[0]
    return pl.pallas_call(
        functools.partial(_tailf_body, float(n)),
        grid=(_row_grid(n, r),),
        in_specs=[_bspec(r, C), _bspec(2, C, True), _bspec(r, C), _bspec(r, C),
                  _bspec(C, C, True), _bspec(1, C, True), _bspec(C, C, True),
                  _bspec(1, C, True)],
        out_specs=[_bspec(r, C), _bspec(r, C)],
        out_shape=[jax.ShapeDtypeStruct((n, C), jnp.float32),
                   jax.ShapeDtypeStruct((n, C), jnp.float32)],
        compiler_params=pltpu.CompilerParams(dimension_semantics=("arbitrary",)),
    )(y4, st, x2, x, wpi, bpi.reshape(1, -1), wtop, b1.reshape(1, -1))


def _tailp_body(nf, y4_ref, st_in, x2_ref, wlg_ref, blg_ref, loss_ref, acc):
    g = pl.program_id(0)
    vp = jax.nn.relu(_norm(y4_ref[...], st_in[...], nf) + x2_ref[...])
    logits = jnp.sum(vp * wlg_ref[...], axis=1) + blg_ref[0, 0]
    t = -logits
    sp = jnp.maximum(t, 0.0) + jnp.log(1.0 + jnp.exp(-jnp.abs(t)))

    @pl.when(g == 0)
    def _():
        acc[...] = jnp.zeros_like(acc)

    acc[...] += jnp.sum(sp).reshape(1, 1)
    loss_ref[...] = (acc[...] + (TOTAL - nf) * jnp.log(2.0)) / TOTAL


def _tailp(y4, st, x2, wlg, blg, r):
    n = y4.shape[0]
    return pl.pallas_call(
        functools.partial(_tailp_body, float(n)),
        grid=(_row_grid(n, r),),
        in_specs=[_bspec(r, C), _bspec(2, C, True), _bspec(r, C),
                  _bspec(1, C, True), _bspec(1, 1, True)],
        out_specs=[_bspec(1, 1, True)],
        out_shape=[jax.ShapeDtypeStruct((1, 1), jnp.float32)],
        scratch_shapes=[pltpu.VMEM((1, 1), jnp.float32)],
        compiler_params=pltpu.CompilerParams(dimension_semantics=("arbitrary",)),
    )(y4, st, x2, wlg.reshape(1, -1), blg.reshape(1, 1))[0]


def _vchain(x, p, wpi=None, bpi=None, wtop=None, b1=None, wlg=None, blg=None, r=1000):
    y1, s1 = _mm(x, p['v1_W1'], p['v1_b1'], r)
    y2, s2 = _nrm(y1, s1, p['v1_W2'], p['v1_b2'], r)
    x2, y3, s3 = _nrmres(y2, s2, x, p['v2_W1'], p['v2_b1'], r)
    y4, s4 = _nrm(y3, s3, p['v2_W2'], p['v2_b2'], r)
    if wlg is None:
        return _tailf(y4, s4, x2, x, wpi, bpi, wtop, b1, r)
    return _tailp(y4, s4, x2, wlg, blg, r)


def _rank_body(pres_ref, rank_ref, nd_ref):
    p = jnp.minimum(pres_ref[0] + pres_ref[1], 1).astype(jnp.float32)
    ii = jax.lax.broadcasted_iota(jnp.int32, (C, C), 0)
    jj = jax.lax.broadcasted_iota(jnp.int32, (C, C), 1)
    mstrict = (ii < jj).astype(jnp.float32)
    ones = jnp.ones((C, C), jnp.float32)
    in_row = jnp.dot(p, mstrict, preferred_element_type=jnp.float32)
    ctot = jnp.dot(p, ones, preferred_element_type=jnp.float32)
    c = ctot
    for s in (1, 2, 4, 8, 16, 32, 64, 128, 256, 512, 1024, 2048):
        c = c + jnp.concatenate([jnp.zeros((s, C), jnp.float32), c[:-s]], axis=0)
    offs = c - ctot
    rank_ref[...] = (in_row + offs).astype(jnp.int32)
    nd_ref[...] = c[4095:4096, 0:1].astype(jnp.int32)


def _rank(pres2):
    return pl.pallas_call(
        _rank_body,
        in_specs=[pl.BlockSpec((2, 4096, C), lambda: (0, 0, 0))],
        out_specs=[pl.BlockSpec((4096, C), lambda: (0, 0)),
                   pl.BlockSpec((1, 1), lambda: (0, 0))],
        out_shape=[jax.ShapeDtypeStruct((4096, C), jnp.int32),
                   jax.ShapeDtypeStruct((1, 1), jnp.int32)],
    )(pres2)


def _q1_body(dsum_ref, dcnt_ref, nd_ref, w_ref, b_ref, pp_ref, st_ref, acc):
    g = pl.program_id(0)
    r = dsum_ref.shape[0]
    cnt = jnp.clip(dcnt_ref[...], 1.0, None)
    down = dsum_ref[...] / cnt
    pp = _lrelu(jnp.dot(down, w_ref[...], preferred_element_type=jnp.float32) + b_ref[...])
    pp_ref[...] = pp
    rows = jax.lax.broadcasted_iota(jnp.int32, (r, 1), 0) + g * r
    mask = (rows < nd_ref[0]).astype(jnp.float32)
    pm = pp * mask

    @pl.when(g == 0)
    def _():
        acc[...] = jnp.zeros_like(acc)

    acc[...] += jnp.stack([jnp.sum(pm, axis=0), jnp.sum(pp * pm, axis=0)])
    st_ref[...] = acc[...]


def _q1(dsum, dcnt, nd, w, b, r):
    n = dsum.shape[0]
    cout = w.shape[1]
    return pl.pallas_call(
        _q1_body,
        grid=(_row_grid(n, r),),
        in_specs=[_bspec(r, C), _bspec(r, 1),
                  pl.BlockSpec(memory_space=pltpu.SMEM),
                  _bspec(C, cout, True), _bspec(1, cout, True)],
        out_specs=[_bspec(r, cout), _bspec(2, cout, True)],
        out_shape=[jax.ShapeDtypeStruct((n, cout), jnp.float32),
                   jax.ShapeDtypeStruct((2, cout), jnp.float32)],
        scratch_shapes=[pltpu.VMEM((2, cout), jnp.float32)],
        compiler_params=pltpu.CompilerParams(dimension_semantics=("arbitrary",)),
    )(dsum, dcnt, nd, w, b.reshape(1, -1))


def _normm(x, st, ndf):
    m = st[0:1, :] / ndf
    v = st[1:2, :] / ndf - m * m
    return (x - m) * jax.lax.rsqrt(v + EPS)


def _q2_body(x_ref, st_in, nd_ref, w_ref, b_ref, o_ref, st_ref, acc):
    g = pl.program_id(0)
    r = x_ref.shape[0]
    ndf = nd_ref[0].astype(jnp.float32)
    xn = _normm(x_ref[...], st_in[...], ndf)
    o = _lrelu(jnp.dot(xn, w_ref[...], preferred_element_type=jnp.float32) + b_ref[...])
    o_ref[...] = o
    rows = jax.lax.broadcasted_iota(jnp.int32, (r, 1), 0) + g * r
    mask = (rows < nd_ref[0]).astype(jnp.float32)
    om = o * mask

    @pl.when(g == 0)
    def _():
        acc[...] = jnp.zeros_like(acc)

    acc[...] += jnp.stack([jnp.sum(om, axis=0), jnp.sum(o * om, axis=0)])
    st_ref[...] = acc[...]


def _q2(x, st, nd, w, b, r):
    n, cin = x.shape
    cout = w.shape[1]
    return pl.pallas_call(
        _q2_body,
        grid=(_row_grid(n, r),),
        in_specs=[_bspec(r, cin), _bspec(2, cin, True),
                  pl.BlockSpec(memory_space=pltpu.SMEM),
                  _bspec(cin, cout, True), _bspec(1, cout, True)],
        out_specs=[_bspec(r, cout), _bspec(2, cout, True)],
        out_shape=[jax.ShapeDtypeStruct((n, cout), jnp.float32),
                   jax.ShapeDtypeStruct((2, cout), jnp.float32)],
        scratch_shapes=[pltpu.VMEM((2, cout), jnp.float32)],
        compiler_params=pltpu.CompilerParams(dimension_semantics=("arbitrary",)),
    )(x, st, nd, w, b.reshape(1, -1))


def _q3_body(x_ref, st_in, nd_ref, w_ref, b_ref, wbot_ref, b2_ref):
    ndf = nd_ref[0].astype(jnp.float32)
    xn = _normm(x_ref[...], st_in[...], ndf)
    pp3 = _lrelu(jnp.dot(xn, w_ref[...], preferred_element_type=jnp.float32) + b_ref[...])
    b2_ref[...] = jnp.dot(pp3, wbot_ref[...], preferred_element_type=jnp.float32)


def _q3(x, st, nd, w, b, wbot, r):
    n, cin = x.shape
    return pl.pallas_call(
        _q3_body,
        grid=(_row_grid(n, r),),
        in_specs=[_bspec(r, cin), _bspec(2, cin, True),
                  pl.BlockSpec(memory_space=pltpu.SMEM),
                  _bspec(cin, C, True), _bspec(1, C, True), _bspec(C, C, True)],
        out_specs=[_bspec(r, C)],
        out_shape=[jax.ShapeDtypeStruct((n, C), jnp.float32)],
        compiler_params=pltpu.CompilerParams(dimension_semantics=("arbitrary",)),
    )(x, st, nd, w, b.reshape(1, -1), wbot)


def _final_body(acc_ref, w_ref, b_ref, o_ref):
    a = acc_ref[0] + acc_ref[1]
    m = a[:, :C] / jnp.clip(a[:, C:C + 1], 1.0, None)
    o_ref[...] = jnp.dot(m, w_ref[...], preferred_element_type=jnp.float32) + b_ref[...]


def _final(acc2, w, b):
    n = acc2.shape[1]
    return pl.pallas_call(
        _final_body,
        in_specs=[pl.BlockSpec((2, n, 144), lambda: (0, 0, 0)),
                  pl.BlockSpec((C, C), lambda: (0, 0)),
                  pl.BlockSpec((1, C), lambda: (0, 0))],
        out_specs=[pl.BlockSpec((n, C), lambda: (0, 0))],
        out_shape=[jax.ShapeDtypeStruct((n, C), jnp.float32)],
    )(acc2, w, b.reshape(1, -1))


def kernel(features, partial_features, params, coors, coors_inv_last, coors_inv_scale):
    p = params
    feat, A = _vchain(features, p, wpi=p['pi_W'], bpi=p['pi_b'],
                      wtop=p['po_W1'][:C], b1=p['po_b1'])
    loss = _vchain(partial_features, p, wlg=p['lg_W'], blg=p['lg_b'])[0, 0]

    key = ((coors[:, 0] << 18) + ((coors[:, 1] >> 1) << 12)
           + ((coors[:, 2] >> 1) << 6) + (coors[:, 3] >> 1))
    pres = jnp.zeros((2, KEYSPACE), jnp.int32).at[0, key].set(1)
    rank2d, nd2d = _rank(pres.reshape(2, 4096, C))
    inv = rank2d.reshape(-1)[key]
    nd = nd2d.reshape(-1)

    dsum = jnp.zeros((N_LAST, C), jnp.float32).at[inv].add(feat)
    dcnt = jnp.zeros((N_LAST, 1), jnp.float32).at[inv, 0].add(1.0)

    pp1, st1 = _q1(dsum, dcnt, nd, p['pp_W1'], p['pp_b1'], 1000)
    pp2, st2 = _q2(pp1, st1, nd, p['pp_W2'], p['pp_b2'], 1000)
    B2 = _q3(pp2, st2, nd, p['pp_W3'], p['pp_b3'], p['po_W1'][C:], 1000)[0]

    h = _lrelu(A[coors_inv_last] + B2[inv[coors_inv_last]])
    hc = jnp.concatenate([h, jnp.ones((N_POINTS, 1), jnp.float32),
                          jnp.zeros((N_POINTS, 15), jnp.float32)], axis=1)
    acc2 = jnp.zeros((2, N_SCALE, 144), jnp.float32).at[0, coors_inv_scale].add(hc)

    p_fea = _final(acc2, p['po_W2'], p['po_b2'])[0]
    return (p_fea[coors_inv_scale], loss)


# speedup vs baseline: 1.2476x; 1.2476x over previous
import functools
import jax
import jax.numpy as jnp
from jax import lax
from jax.experimental import pallas as pl
from jax.experimental.pallas import tpu as pltpu
from jax.experimental.pallas import tpu_sc as plsc

N_LAST = 50000
N_PARTIAL = 25000
N_POINTS = 100000
N_SCALE = 12500
C = 128
KEYSPACE = 524288
TOTAL = float(2 * 128 * 128 * 16)
EPS = 1e-5


def _row_grid(n, r):
    assert n % r == 0
    return n // r


def _bspec(r, c, const=False):
    if const:
        return pl.BlockSpec((r, c), lambda g: (0, 0))
    return pl.BlockSpec((r, c), lambda g: (g, 0))


def _mm_body(x_ref, w_ref, b_ref, y_ref, st_ref, acc):
    g = pl.program_id(0)
    y = jnp.dot(x_ref[...], w_ref[...], preferred_element_type=jnp.float32) + b_ref[...]
    y_ref[...] = y

    @pl.when(g == 0)
    def _():
        acc[...] = jnp.zeros_like(acc)

    acc[...] += jnp.stack([jnp.sum(y, axis=0), jnp.sum(y * y, axis=0)])
    st_ref[...] = acc[...]


def _mm(x, w, b, r):
    n, cin = x.shape
    cout = w.shape[1]
    return pl.pallas_call(
        _mm_body,
        grid=(_row_grid(n, r),),
        in_specs=[_bspec(r, cin), _bspec(cin, cout, True), _bspec(1, cout, True)],
        out_specs=[_bspec(r, cout), _bspec(2, cout, True)],
        out_shape=[jax.ShapeDtypeStruct((n, cout), jnp.float32),
                   jax.ShapeDtypeStruct((2, cout), jnp.float32)],
        scratch_shapes=[pltpu.VMEM((2, cout), jnp.float32)],
        compiler_params=pltpu.CompilerParams(dimension_semantics=("arbitrary",)),
    )(x, w, b.reshape(1, -1))


def _norm(y, st, nf):
    m = st[0:1, :] * (1.0 / nf)
    v = st[1:2, :] * (1.0 / nf) - m * m
    return (y - m) * jax.lax.rsqrt(v + EPS)


def _nrm_body(nf, y_ref, st_in, w_ref, b_ref, y2_ref, st_ref, acc):
    g = pl.program_id(0)
    z = jax.nn.relu(_norm(y_ref[...], st_in[...], nf))
    y2 = jnp.dot(z, w_ref[...], preferred_element_type=jnp.float32) + b_ref[...]
    y2_ref[...] = y2

    @pl.when(g == 0)
    def _():
        acc[...] = jnp.zeros_like(acc)

    acc[...] += jnp.stack([jnp.sum(y2, axis=0), jnp.sum(y2 * y2, axis=0)])
    st_ref[...] = acc[...]


def _nrm(y, st, w, b, r):
    n, cin = y.shape
    cout = w.shape[1]
    return pl.pallas_call(
        functools.partial(_nrm_body, float(n)),
        grid=(_row_grid(n, r),),
        in_specs=[_bspec(r, cin), _bspec(2, cin, True), _bspec(cin, cout, True),
                  _bspec(1, cout, True)],
        out_specs=[_bspec(r, cout), _bspec(2, cout, True)],
        out_shape=[jax.ShapeDtypeStruct((n, cout), jnp.float32),
                   jax.ShapeDtypeStruct((2, cout), jnp.float32)],
        scratch_shapes=[pltpu.VMEM((2, cout), jnp.float32)],
        compiler_params=pltpu.CompilerParams(dimension_semantics=("arbitrary",)),
    )(y, st, w, b.reshape(1, -1))


def _nrmres_body(nf, y_ref, st_in, xres_ref, w_ref, b_ref, x2_ref, y2_ref, st_ref, acc):
    g = pl.program_id(0)
    x2 = jax.nn.relu(_norm(y_ref[...], st_in[...], nf) + xres_ref[...])
    x2_ref[...] = x2
    y2 = jnp.dot(x2, w_ref[...], preferred_element_type=jnp.float32) + b_ref[...]
    y2_ref[...] = y2

    @pl.when(g == 0)
    def _():
        acc[...] = jnp.zeros_like(acc)

    acc[...] += jnp.stack([jnp.sum(y2, axis=0), jnp.sum(y2 * y2, axis=0)])
    st_ref[...] = acc[...]


def _nrmres(y, st, xres, w, b, r):
    n, cin = y.shape
    cout = w.shape[1]
    return pl.pallas_call(
        functools.partial(_nrmres_body, float(n)),
        grid=(_row_grid(n, r),),
        in_specs=[_bspec(r, cin), _bspec(2, cin, True), _bspec(r, cin),
                  _bspec(cin, cout, True), _bspec(1, cout, True)],
        out_specs=[_bspec(r, cin), _bspec(r, cout), _bspec(2, cout, True)],
        out_shape=[jax.ShapeDtypeStruct((n, cin), jnp.float32),
                   jax.ShapeDtypeStruct((n, cout), jnp.float32),
                   jax.ShapeDtypeStruct((2, cout), jnp.float32)],
        scratch_shapes=[pltpu.VMEM((2, cout), jnp.float32)],
        compiler_params=pltpu.CompilerParams(dimension_semantics=("arbitrary",)),
    )(y, st, xres, w, b.reshape(1, -1))


def _lrelu(x):
    return jnp.where(x > 0, x, 0.1 * x)


def _tailf_body(nf, y4_ref, st_in, x2_ref, x_ref, wpi_ref, bpi_ref, wtop_ref,
                b1_ref, feat_ref, a_ref):
    v = jax.nn.relu(_norm(y4_ref[...], st_in[...], nf) + x2_ref[...])
    feat = x_ref[...] + v
    feat_ref[...] = feat
    t = _lrelu(jnp.dot(feat, wpi_ref[...], preferred_element_type=jnp.float32)
               + bpi_ref[...])
    a_ref[...] = jnp.dot(t, wtop_ref[...], preferred_element_type=jnp.float32) + b1_ref[...]


def _tailf(y4, st, x2, x, wpi, bpi, wtop, b1, r):
    n = y4.shape[0]
    return pl.pallas_call(
        functools.partial(_tailf_body, float(n)),
        grid=(_row_grid(n, r),),
        in_specs=[_bspec(r, C), _bspec(2, C, True), _bspec(r, C), _bspec(r, C),
                  _bspec(C, C, True), _bspec(1, C, True), _bspec(C, C, True),
                  _bspec(1, C, True)],
        out_specs=[_bspec(r, C), _bspec(r, C)],
        out_shape=[jax.ShapeDtypeStruct((n, C), jnp.float32),
                   jax.ShapeDtypeStruct((n, C), jnp.float32)],
        compiler_params=pltpu.CompilerParams(dimension_semantics=("arbitrary",)),
    )(y4, st, x2, x, wpi, bpi.reshape(1, -1), wtop, b1.reshape(1, -1))


def _tailp_body(nf, y4_ref, st_in, x2_ref, wlg_ref, blg_ref, loss_ref, acc):
    g = pl.program_id(0)
    vp = jax.nn.relu(_norm(y4_ref[...], st_in[...], nf) + x2_ref[...])
    logits = jnp.sum(vp * wlg_ref[...], axis=1) + blg_ref[0, 0]
    t = -logits
    sp = jnp.maximum(t, 0.0) + jnp.log(1.0 + jnp.exp(-jnp.abs(t)))

    @pl.when(g == 0)
    def _():
        acc[...] = jnp.zeros_like(acc)

    acc[...] += jnp.sum(sp).reshape(1, 1)
    loss_ref[...] = (acc[...] + (TOTAL - nf) * jnp.log(2.0)) / TOTAL


def _tailp(y4, st, x2, wlg, blg, r):
    n = y4.shape[0]
    return pl.pallas_call(
        functools.partial(_tailp_body, float(n)),
        grid=(_row_grid(n, r),),
        in_specs=[_bspec(r, C), _bspec(2, C, True), _bspec(r, C),
                  _bspec(1, C, True), _bspec(1, 1, True)],
        out_specs=[_bspec(1, 1, True)],
        out_shape=[jax.ShapeDtypeStruct((1, 1), jnp.float32)],
        scratch_shapes=[pltpu.VMEM((1, 1), jnp.float32)],
        compiler_params=pltpu.CompilerParams(dimension_semantics=("arbitrary",)),
    )(y4, st, x2, wlg.reshape(1, -1), blg.reshape(1, 1))[0]


def _vchain(x, p, wpi=None, bpi=None, wtop=None, b1=None, wlg=None, blg=None, r=1000):
    y1, s1 = _mm(x, p['v1_W1'], p['v1_b1'], r)
    y2, s2 = _nrm(y1, s1, p['v1_W2'], p['v1_b2'], r)
    x2, y3, s3 = _nrmres(y2, s2, x, p['v2_W1'], p['v2_b1'], r)
    y4, s4 = _nrm(y3, s3, p['v2_W2'], p['v2_b2'], r)
    if wlg is None:
        return _tailf(y4, s4, x2, x, wpi, bpi, wtop, b1, r)
    return _tailp(y4, s4, x2, wlg, blg, r)


def _rank_body(pres_ref, rank_ref, nd_ref):
    p = jnp.minimum(pres_ref[0] + pres_ref[1], 1).astype(jnp.float32)
    ii = jax.lax.broadcasted_iota(jnp.int32, (C, C), 0)
    jj = jax.lax.broadcasted_iota(jnp.int32, (C, C), 1)
    mstrict = (ii < jj).astype(jnp.float32)
    ones = jnp.ones((C, C), jnp.float32)
    in_row = jnp.dot(p, mstrict, preferred_element_type=jnp.float32)
    ctot = jnp.dot(p, ones, preferred_element_type=jnp.float32)
    c = ctot
    for s in (1, 2, 4, 8, 16, 32, 64, 128, 256, 512, 1024, 2048):
        c = c + jnp.concatenate([jnp.zeros((s, C), jnp.float32), c[:-s]], axis=0)
    offs = c - ctot
    rank_ref[...] = (in_row + offs).astype(jnp.int32)
    nd_ref[...] = c[4095:4096, 0:1].astype(jnp.int32)


def _rank(pres2):
    return pl.pallas_call(
        _rank_body,
        in_specs=[pl.BlockSpec((2, 4096, C), lambda: (0, 0, 0))],
        out_specs=[pl.BlockSpec((4096, C), lambda: (0, 0)),
                   pl.BlockSpec((1, 1), lambda: (0, 0))],
        out_shape=[jax.ShapeDtypeStruct((4096, C), jnp.int32),
                   jax.ShapeDtypeStruct((1, 1), jnp.int32)],
    )(pres2)


def _q1_body(dsum_ref, dcnt_ref, nd_ref, w_ref, b_ref, pp_ref, st_ref, acc):
    g = pl.program_id(0)
    r = dsum_ref.shape[0]
    cnt = jnp.clip(dcnt_ref[...], 1.0, None)
    down = dsum_ref[...] / cnt
    pp = _lrelu(jnp.dot(down, w_ref[...], preferred_element_type=jnp.float32) + b_ref[...])
    pp_ref[...] = pp
    rows = jax.lax.broadcasted_iota(jnp.int32, (r, 1), 0) + g * r
    mask = (rows < nd_ref[0]).astype(jnp.float32)
    pm = pp * mask

    @pl.when(g == 0)
    def _():
        acc[...] = jnp.zeros_like(acc)

    acc[...] += jnp.stack([jnp.sum(pm, axis=0), jnp.sum(pp * pm, axis=0)])
    st_ref[...] = acc[...]


def _q1(dsum, dcnt, nd, w, b, r):
    n = dsum.shape[0]
    cout = w.shape[1]
    return pl.pallas_call(
        _q1_body,
        grid=(_row_grid(n, r),),
        in_specs=[_bspec(r, C), _bspec(r, 1),
                  pl.BlockSpec(memory_space=pltpu.SMEM),
                  _bspec(C, cout, True), _bspec(1, cout, True)],
        out_specs=[_bspec(r, cout), _bspec(2, cout, True)],
        out_shape=[jax.ShapeDtypeStruct((n, cout), jnp.float32),
                   jax.ShapeDtypeStruct((2, cout), jnp.float32)],
        scratch_shapes=[pltpu.VMEM((2, cout), jnp.float32)],
        compiler_params=pltpu.CompilerParams(dimension_semantics=("arbitrary",)),
    )(dsum, dcnt, nd, w, b.reshape(1, -1))


def _normm(x, st, ndf):
    m = st[0:1, :] / ndf
    v = st[1:2, :] / ndf - m * m
    return (x - m) * jax.lax.rsqrt(v + EPS)


def _q2_body(x_ref, st_in, nd_ref, w_ref, b_ref, o_ref, st_ref, acc):
    g = pl.program_id(0)
    r = x_ref.shape[0]
    ndf = nd_ref[0].astype(jnp.float32)
    xn = _normm(x_ref[...], st_in[...], ndf)
    o = _lrelu(jnp.dot(xn, w_ref[...], preferred_element_type=jnp.float32) + b_ref[...])
    o_ref[...] = o
    rows = jax.lax.broadcasted_iota(jnp.int32, (r, 1), 0) + g * r
    mask = (rows < nd_ref[0]).astype(jnp.float32)
    om = o * mask

    @pl.when(g == 0)
    def _():
        acc[...] = jnp.zeros_like(acc)

    acc[...] += jnp.stack([jnp.sum(om, axis=0), jnp.sum(o * om, axis=0)])
    st_ref[...] = acc[...]


def _q2(x, st, nd, w, b, r):
    n, cin = x.shape
    cout = w.shape[1]
    return pl.pallas_call(
        _q2_body,
        grid=(_row_grid(n, r),),
        in_specs=[_bspec(r, cin), _bspec(2, cin, True),
                  pl.BlockSpec(memory_space=pltpu.SMEM),
                  _bspec(cin, cout, True), _bspec(1, cout, True)],
        out_specs=[_bspec(r, cout), _bspec(2, cout, True)],
        out_shape=[jax.ShapeDtypeStruct((n, cout), jnp.float32),
                   jax.ShapeDtypeStruct((2, cout), jnp.float32)],
        scratch_shapes=[pltpu.VMEM((2, cout), jnp.float32)],
        compiler_params=pltpu.CompilerParams(dimension_semantics=("arbitrary",)),
    )(x, st, nd, w, b.reshape(1, -1))


def _q3_body(x_ref, st_in, nd_ref, w_ref, b_ref, wbot_ref, b2_ref):
    ndf = nd_ref[0].astype(jnp.float32)
    xn = _normm(x_ref[...], st_in[...], ndf)
    pp3 = _lrelu(jnp.dot(xn, w_ref[...], preferred_element_type=jnp.float32) + b_ref[...])
    b2_ref[...] = jnp.dot(pp3, wbot_ref[...], preferred_element_type=jnp.float32)


def _q3(x, st, nd, w, b, wbot, r):
    n, cin = x.shape
    return pl.pallas_call(
        _q3_body,
        grid=(_row_grid(n, r),),
        in_specs=[_bspec(r, cin), _bspec(2, cin, True),
                  pl.BlockSpec(memory_space=pltpu.SMEM),
                  _bspec(cin, C, True), _bspec(1, C, True), _bspec(C, C, True)],
        out_specs=[_bspec(r, C)],
        out_shape=[jax.ShapeDtypeStruct((n, C), jnp.float32)],
        compiler_params=pltpu.CompilerParams(dimension_semantics=("arbitrary",)),
    )(x, st, nd, w, b.reshape(1, -1), wbot)


def _final_body(acc_ref, cnt_ref, w_ref, b_ref, o_ref):
    a = jnp.concatenate([acc_ref[0, :HALF], acc_ref[1, 1:N_SCALE - HALF + 1]],
                        axis=0)
    cnt = jnp.concatenate([cnt_ref[0, :HALF, 0:1],
                           cnt_ref[1, 1:N_SCALE - HALF + 1, 0:1]], axis=0)
    m = a / jnp.clip(cnt, 1.0, None)
    o_ref[...] = jnp.dot(m, w_ref[...], preferred_element_type=jnp.float32) + b_ref[...]


def _final(acc2, cnt2, w, b):
    return pl.pallas_call(
        _final_body,
        in_specs=[pl.BlockSpec((2, ACC_R, C), lambda: (0, 0, 0)),
                  pl.BlockSpec((2, ACC_R, C), lambda: (0, 0, 0)),
                  pl.BlockSpec((C, C), lambda: (0, 0)),
                  pl.BlockSpec((1, C), lambda: (0, 0))],
        out_specs=[pl.BlockSpec((N_SCALE, C), lambda: (0, 0))],
        out_shape=[jax.ShapeDtypeStruct((N_SCALE, C), jnp.float32)],
    )(acc2, cnt2, w, b.reshape(1, -1))


HALF = 6272
ACC_R = 6400
_CH = 80
_NCHUNK = N_POINTS // _CH


def _sc_mesh():
    return plsc.VectorSubcoreMesh(core_axis_name="c", subcore_axis_name="s")


def _zero_fill(buf, rows, width):
    @pl.loop(0, rows)
    def zrow(r):
        zero16 = jnp.zeros((16,), jnp.float32)
        for j in range(width // 16):
            buf[r, pl.ds(j * 16, 16)] = zero16


def _seg_idx(cid, cis_v, idx_v):
    @pl.when(cid == 0)
    def _():
        for j in range(_CH // 16):
            v = cis_v[pl.ds(j * 16, 16)]
            idx_v[pl.ds(j * 16, 16)] = jnp.minimum(v, HALF)

    @pl.when(cid == 1)
    def _():
        for j in range(_CH // 16):
            v = cis_v[pl.ds(j * 16, 16)]
            idx_v[pl.ds(j * 16, 16)] = jnp.maximum(v - (HALF - 1), 0)


@functools.partial(
    pl.kernel,
    mesh=_sc_mesh(),
    out_type=jax.ShapeDtypeStruct((2, ACC_R, C), jnp.float32),
    scratch_types=[
        pltpu.VMEM((_CH,), jnp.int32),
        pltpu.VMEM((_CH,), jnp.int32),
        pltpu.VMEM((_CH,), jnp.int32),
        pltpu.VMEM((_CH,), jnp.int32),
        pltpu.VMEM((_CH, C), jnp.float32),
        pltpu.VMEM((_CH, C), jnp.float32),
        pltpu.VMEM((_CH, C), jnp.float32),
        pltpu.VMEM((_CH, C), jnp.float32),
        pltpu.VMEM_SHARED((ACC_R, C), jnp.float32),
        pltpu.SemaphoreType.DMA,
        pltpu.SemaphoreType.DMA,
    ],
)
def _sc_main(a_hbm, b2_hbm, cil_hbm, invc_hbm, cis_hbm, acc_out,
             cil_v, invc_v, cis_v, idx_v, ra_v, rb_v, y_v, z_v, acc_sh,
             sem1, sem2):
    cid = lax.axis_index("c")
    sid = lax.axis_index("s")

    _zero_fill(z_v, _CH, C)

    @pl.loop(0, 5)
    def zcp(i):
        pltpu.sync_copy(z_v, acc_sh.at[pl.ds(sid * 400 + i * 80, 80)])

    plsc.subcore_barrier()

    @pl.loop(0, (_NCHUNK + 15) // 16)
    def chunk(i):
        k = sid + i * 16

        @pl.when(k < _NCHUNK)
        def _():
            base = k * _CH
            pltpu.sync_copy(cil_hbm.at[pl.ds(base, _CH)], cil_v)
            pltpu.sync_copy(invc_hbm.at[pl.ds(base, _CH)], invc_v)
            pltpu.sync_copy(cis_hbm.at[pl.ds(base, _CH)], cis_v)
            cp_a = pltpu.async_copy(a_hbm.at[cil_v], ra_v, sem1)
            cp_b = pltpu.async_copy(b2_hbm.at[invc_v], rb_v, sem2)
            _seg_idx(cid, cis_v, idx_v)
            cp_a.wait()
            cp_b.wait()

            @pl.loop(0, _CH)
            def row(r):
                for j in range(C // 16):
                    a = ra_v[r, pl.ds(j * 16, 16)] + rb_v[r, pl.ds(j * 16, 16)]
                    y_v[r, pl.ds(j * 16, 16)] = jnp.maximum(a, 0.1 * a)

            pltpu.sync_copy(y_v, acc_sh.at[idx_v], add=True)

    plsc.subcore_barrier()
    pltpu.sync_copy(acc_sh.at[pl.ds(sid * 400, 400)],
                    acc_out.at[cid, pl.ds(sid * 400, 400)])


@functools.partial(
    pl.kernel,
    mesh=_sc_mesh(),
    out_type=jax.ShapeDtypeStruct((2, ACC_R, C), jnp.float32),
    scratch_types=[
        pltpu.VMEM((_CH,), jnp.int32),
        pltpu.VMEM((_CH,), jnp.int32),
        pltpu.VMEM((_CH, C), jnp.float32),
        pltpu.VMEM((_CH, C), jnp.float32),
        pltpu.VMEM_SHARED((ACC_R, C), jnp.float32),
    ],
)
def _sc_cnt(cis_hbm, cnt_out, cis_v, idx_v, ones_v, z_v, acc_sh):
    cid = lax.axis_index("c")
    sid = lax.axis_index("s")

    _zero_fill(z_v, _CH, C)

    @pl.loop(0, 5)
    def zcp(i):
        pltpu.sync_copy(z_v, acc_sh.at[pl.ds(sid * 400 + i * 80, 80)])

    @pl.loop(0, _CH)
    def fill1(r):
        one16 = jnp.ones((16,), jnp.float32)
        for j in range(C // 16):
            ones_v[r, pl.ds(j * 16, 16)] = one16

    plsc.subcore_barrier()

    @pl.loop(0, (_NCHUNK + 15) // 16)
    def chunk(i):
        k = sid + i * 16

        @pl.when(k < _NCHUNK)
        def _():
            pltpu.sync_copy(cis_hbm.at[pl.ds(k * _CH, _CH)], cis_v)
            _seg_idx(cid, cis_v, idx_v)
            pltpu.sync_copy(ones_v, acc_sh.at[idx_v], add=True)

    plsc.subcore_barrier()
    pltpu.sync_copy(acc_sh.at[pl.ds(sid * 400, 400)],
                    cnt_out.at[cid, pl.ds(sid * 400, 400)])


@functools.partial(
    pl.kernel,
    mesh=_sc_mesh(),
    out_type=jax.ShapeDtypeStruct((N_POINTS, C), jnp.float32),
    scratch_types=[
        pltpu.VMEM((_CH,), jnp.int32),
        pltpu.VMEM((_CH, C), jnp.float32),
        pltpu.SemaphoreType.DMA,
    ],
)
def _sc_fgather(pfea_hbm, cis_hbm, out_hbm, cis_v, rows_v, sem):
    cid = lax.axis_index("c")
    sid = lax.axis_index("s")
    w = sid * 2 + cid

    @pl.loop(0, (_NCHUNK + 31) // 32)
    def chunk(i):
        k = w + i * 32

        @pl.when(k < _NCHUNK)
        def _():
            base = k * _CH
            pltpu.sync_copy(cis_hbm.at[pl.ds(base, _CH)], cis_v)
            pltpu.async_copy(pfea_hbm.at[cis_v], rows_v, sem).wait()
            pltpu.sync_copy(rows_v, out_hbm.at[pl.ds(base, _CH)])


def kernel(features, partial_features, params, coors, coors_inv_last, coors_inv_scale):
    p = params
    feat, A = _vchain(features, p, wpi=p['pi_W'], bpi=p['pi_b'],
                      wtop=p['po_W1'][:C], b1=p['po_b1'])
    loss = _vchain(partial_features, p, wlg=p['lg_W'], blg=p['lg_b'])[0, 0]

    key = ((coors[:, 0] << 18) + ((coors[:, 1] >> 1) << 12)
           + ((coors[:, 2] >> 1) << 6) + (coors[:, 3] >> 1))
    pres = jnp.zeros((2, KEYSPACE), jnp.int32).at[0, key].set(1)
    rank2d, nd2d = _rank(pres.reshape(2, 4096, C))
    inv = rank2d.reshape(-1)[key]
    nd = nd2d.reshape(-1)

    dsum = jnp.zeros((N_LAST, C), jnp.float32).at[inv].add(feat)
    dcnt = jnp.zeros((N_LAST, 1), jnp.float32).at[inv, 0].add(1.0)

    pp1, st1 = _q1(dsum, dcnt, nd, p['pp_W1'], p['pp_b1'], 1000)
    pp2, st2 = _q2(pp1, st1, nd, p['pp_W2'], p['pp_b2'], 1000)
    B2 = _q3(pp2, st2, nd, p['pp_W3'], p['pp_b3'], p['po_W1'][C:], 1000)[0]

    invc = inv[coors_inv_last]
    acc2 = _sc_main(A, B2, coors_inv_last, invc, coors_inv_scale)
    cnt2 = _sc_cnt(coors_inv_scale)
    p_fea = _final(acc2, cnt2, p['po_W2'], p['po_b2'])[0]
    return (_sc_fgather(p_fea, coors_inv_scale), loss)


# speedup vs baseline: 1.4307x; 1.1468x over previous
import functools
import jax
import jax.numpy as jnp
from jax import lax
from jax.experimental import pallas as pl
from jax.experimental.pallas import tpu as pltpu
from jax.experimental.pallas import tpu_sc as plsc

N_LAST = 50000
N_PARTIAL = 25000
N_POINTS = 100000
N_SCALE = 12500
C = 128
KEYSPACE = 524288
TOTAL = float(2 * 128 * 128 * 16)
EPS = 1e-5


def _row_grid(n, r):
    assert n % r == 0
    return n // r


def _bspec(r, c, const=False):
    if const:
        return pl.BlockSpec((r, c), lambda g: (0, 0))
    return pl.BlockSpec((r, c), lambda g: (g, 0))


def _mm_body(x_ref, w_ref, b_ref, y_ref, st_ref, acc):
    g = pl.program_id(0)
    y = jnp.dot(x_ref[...], w_ref[...], preferred_element_type=jnp.float32) + b_ref[...]
    y_ref[...] = y

    @pl.when(g == 0)
    def _():
        acc[...] = jnp.zeros_like(acc)

    acc[...] += jnp.stack([jnp.sum(y, axis=0), jnp.sum(y * y, axis=0)])
    st_ref[...] = acc[...]


def _mm(x, w, b, r):
    n, cin = x.shape
    cout = w.shape[1]
    return pl.pallas_call(
        _mm_body,
        grid=(_row_grid(n, r),),
        in_specs=[_bspec(r, cin), _bspec(cin, cout, True), _bspec(1, cout, True)],
        out_specs=[_bspec(r, cout), _bspec(2, cout, True)],
        out_shape=[jax.ShapeDtypeStruct((n, cout), jnp.float32),
                   jax.ShapeDtypeStruct((2, cout), jnp.float32)],
        scratch_shapes=[pltpu.VMEM((2, cout), jnp.float32)],
        compiler_params=pltpu.CompilerParams(dimension_semantics=("arbitrary",)),
    )(x, w, b.reshape(1, -1))


def _norm(y, st, nf):
    m = st[0:1, :] * (1.0 / nf)
    v = st[1:2, :] * (1.0 / nf) - m * m
    return (y - m) * jax.lax.rsqrt(v + EPS)


def _nrm_body(nf, y_ref, st_in, w_ref, b_ref, y2_ref, st_ref, acc):
    g = pl.program_id(0)
    z = jax.nn.relu(_norm(y_ref[...], st_in[...], nf))
    y2 = jnp.dot(z, w_ref[...], preferred_element_type=jnp.float32) + b_ref[...]
    y2_ref[...] = y2

    @pl.when(g == 0)
    def _():
        acc[...] = jnp.zeros_like(acc)

    acc[...] += jnp.stack([jnp.sum(y2, axis=0), jnp.sum(y2 * y2, axis=0)])
    st_ref[...] = acc[...]


def _nrm(y, st, w, b, r):
    n, cin = y.shape
    cout = w.shape[1]
    return pl.pallas_call(
        functools.partial(_nrm_body, float(n)),
        grid=(_row_grid(n, r),),
        in_specs=[_bspec(r, cin), _bspec(2, cin, True), _bspec(cin, cout, True),
                  _bspec(1, cout, True)],
        out_specs=[_bspec(r, cout), _bspec(2, cout, True)],
        out_shape=[jax.ShapeDtypeStruct((n, cout), jnp.float32),
                   jax.ShapeDtypeStruct((2, cout), jnp.float32)],
        scratch_shapes=[pltpu.VMEM((2, cout), jnp.float32)],
        compiler_params=pltpu.CompilerParams(dimension_semantics=("arbitrary",)),
    )(y, st, w, b.reshape(1, -1))


def _nrmres_body(nf, y_ref, st_in, xres_ref, w_ref, b_ref, x2_ref, y2_ref, st_ref, acc):
    g = pl.program_id(0)
    x2 = jax.nn.relu(_norm(y_ref[...], st_in[...], nf) + xres_ref[...])
    x2_ref[...] = x2
    y2 = jnp.dot(x2, w_ref[...], preferred_element_type=jnp.float32) + b_ref[...]
    y2_ref[...] = y2

    @pl.when(g == 0)
    def _():
        acc[...] = jnp.zeros_like(acc)

    acc[...] += jnp.stack([jnp.sum(y2, axis=0), jnp.sum(y2 * y2, axis=0)])
    st_ref[...] = acc[...]


def _nrmres(y, st, xres, w, b, r):
    n, cin = y.shape
    cout = w.shape[1]
    return pl.pallas_call(
        functools.partial(_nrmres_body, float(n)),
        grid=(_row_grid(n, r),),
        in_specs=[_bspec(r, cin), _bspec(2, cin, True), _bspec(r, cin),
                  _bspec(cin, cout, True), _bspec(1, cout, True)],
        out_specs=[_bspec(r, cin), _bspec(r, cout), _bspec(2, cout, True)],
        out_shape=[jax.ShapeDtypeStruct((n, cin), jnp.float32),
                   jax.ShapeDtypeStruct((n, cout), jnp.float32),
                   jax.ShapeDtypeStruct((2, cout), jnp.float32)],
        scratch_shapes=[pltpu.VMEM((2, cout), jnp.float32)],
        compiler_params=pltpu.CompilerParams(dimension_semantics=("arbitrary",)),
    )(y, st, xres, w, b.reshape(1, -1))


def _lrelu(x):
    return jnp.where(x > 0, x, 0.1 * x)


def _tailf_body(nf, y4_ref, st_in, x2_ref, x_ref, wpi_ref, bpi_ref, wtop_ref,
                b1_ref, feat_ref, a_ref):
    v = jax.nn.relu(_norm(y4_ref[...], st_in[...], nf) + x2_ref[...])
    feat = x_ref[...] + v
    feat_ref[...] = feat
    t = _lrelu(jnp.dot(feat, wpi_ref[...], preferred_element_type=jnp.float32)
               + bpi_ref[...])
    a_ref[...] = jnp.dot(t, wtop_ref[...], preferred_element_type=jnp.float32) + b1_ref[...]


def _tailf(y4, st, x2, x, wpi, bpi, wtop, b1, r):
    n = y4.shape[0]
    return pl.pallas_call(
        functools.partial(_tailf_body, float(n)),
        grid=(_row_grid(n, r),),
        in_specs=[_bspec(r, C), _bspec(2, C, True), _bspec(r, C), _bspec(r, C),
                  _bspec(C, C, True), _bspec(1, C, True), _bspec(C, C, True),
                  _bspec(1, C, True)],
        out_specs=[_bspec(r, C), _bspec(r, C)],
        out_shape=[jax.ShapeDtypeStruct((n, C), jnp.float32),
                   jax.ShapeDtypeStruct((n, C), jnp.float32)],
        compiler_params=pltpu.CompilerParams(dimension_semantics=("arbitrary",)),
    )(y4, st, x2, x, wpi, bpi.reshape(1, -1), wtop, b1.reshape(1, -1))


def _tailp_body(nf, y4_ref, st_in, x2_ref, wlg_ref, blg_ref, loss_ref, acc):
    g = pl.program_id(0)
    vp = jax.nn.relu(_norm(y4_ref[...], st_in[...], nf) + x2_ref[...])
    logits = jnp.sum(vp * wlg_ref[...], axis=1) + blg_ref[0, 0]
    t = -logits
    sp = jnp.maximum(t, 0.0) + jnp.log(1.0 + jnp.exp(-jnp.abs(t)))

    @pl.when(g == 0)
    def _():
        acc[...] = jnp.zeros_like(acc)

    acc[...] += jnp.sum(sp).reshape(1, 1)
    loss_ref[...] = (acc[...] + (TOTAL - nf) * jnp.log(2.0)) / TOTAL


def _tailp(y4, st, x2, wlg, blg, r):
    n = y4.shape[0]
    return pl.pallas_call(
        functools.partial(_tailp_body, float(n)),
        grid=(_row_grid(n, r),),
        in_specs=[_bspec(r, C), _bspec(2, C, True), _bspec(r, C),
                  _bspec(1, C, True), _bspec(1, 1, True)],
        out_specs=[_bspec(1, 1, True)],
        out_shape=[jax.ShapeDtypeStruct((1, 1), jnp.float32)],
        scratch_shapes=[pltpu.VMEM((1, 1), jnp.float32)],
        compiler_params=pltpu.CompilerParams(dimension_semantics=("arbitrary",)),
    )(y4, st, x2, wlg.reshape(1, -1), blg.reshape(1, 1))[0]


def _vchain(x, p, wpi=None, bpi=None, wtop=None, b1=None, wlg=None, blg=None, r=1000):
    y1, s1 = _mm(x, p['v1_W1'], p['v1_b1'], r)
    y2, s2 = _nrm(y1, s1, p['v1_W2'], p['v1_b2'], r)
    x2, y3, s3 = _nrmres(y2, s2, x, p['v2_W1'], p['v2_b1'], r)
    y4, s4 = _nrm(y3, s3, p['v2_W2'], p['v2_b2'], r)
    if wlg is None:
        return _tailf(y4, s4, x2, x, wpi, bpi, wtop, b1, r)
    return _tailp(y4, s4, x2, wlg, blg, r)


def _rank_body(pres_ref, rank_ref, nd_ref):
    p = jnp.minimum(pres_ref[0] + pres_ref[1], 1).astype(jnp.float32)
    ii = jax.lax.broadcasted_iota(jnp.int32, (C, C), 0)
    jj = jax.lax.broadcasted_iota(jnp.int32, (C, C), 1)
    mstrict = (ii < jj).astype(jnp.float32)
    ones = jnp.ones((C, C), jnp.float32)
    in_row = jnp.dot(p, mstrict, preferred_element_type=jnp.float32)
    ctot = jnp.dot(p, ones, preferred_element_type=jnp.float32)
    c = ctot
    for s in (1, 2, 4, 8, 16, 32, 64, 128, 256, 512, 1024, 2048):
        c = c + jnp.concatenate([jnp.zeros((s, C), jnp.float32), c[:-s]], axis=0)
    offs = c - ctot
    rank_ref[...] = (in_row + offs).astype(jnp.int32)
    nd_ref[...] = c[4095:4096, 0:1].astype(jnp.int32)


def _rank(pres2):
    return pl.pallas_call(
        _rank_body,
        in_specs=[pl.BlockSpec((2, 4096, C), lambda: (0, 0, 0))],
        out_specs=[pl.BlockSpec((4096, C), lambda: (0, 0)),
                   pl.BlockSpec((1, 1), lambda: (0, 0))],
        out_shape=[jax.ShapeDtypeStruct((4096, C), jnp.int32),
                   jax.ShapeDtypeStruct((1, 1), jnp.int32)],
    )(pres2)


def _q1_body(dsum_ref, dcnt_ref, nd_ref, w_ref, b_ref, pp_ref, st_ref, acc):
    g = pl.program_id(0)
    r = dsum_ref.shape[0]
    cnt = jnp.clip(dcnt_ref[...], 1.0, None)
    down = dsum_ref[...] / cnt
    pp = _lrelu(jnp.dot(down, w_ref[...], preferred_element_type=jnp.float32) + b_ref[...])
    pp_ref[...] = pp
    rows = jax.lax.broadcasted_iota(jnp.int32, (r, 1), 0) + g * r
    mask = (rows < nd_ref[0]).astype(jnp.float32)
    pm = pp * mask

    @pl.when(g == 0)
    def _():
        acc[...] = jnp.zeros_like(acc)

    acc[...] += jnp.stack([jnp.sum(pm, axis=0), jnp.sum(pp * pm, axis=0)])
    st_ref[...] = acc[...]


def _q1(dsum, dcnt, nd, w, b, r):
    n = dsum.shape[0]
    cout = w.shape[1]
    return pl.pallas_call(
        _q1_body,
        grid=(_row_grid(n, r),),
        in_specs=[_bspec(r, C), _bspec(r, 1),
                  pl.BlockSpec(memory_space=pltpu.SMEM),
                  _bspec(C, cout, True), _bspec(1, cout, True)],
        out_specs=[_bspec(r, cout), _bspec(2, cout, True)],
        out_shape=[jax.ShapeDtypeStruct((n, cout), jnp.float32),
                   jax.ShapeDtypeStruct((2, cout), jnp.float32)],
        scratch_shapes=[pltpu.VMEM((2, cout), jnp.float32)],
        compiler_params=pltpu.CompilerParams(dimension_semantics=("arbitrary",)),
    )(dsum, dcnt, nd, w, b.reshape(1, -1))


def _normm(x, st, ndf):
    m = st[0:1, :] / ndf
    v = st[1:2, :] / ndf - m * m
    return (x - m) * jax.lax.rsqrt(v + EPS)


def _q2_body(x_ref, st_in, nd_ref, w_ref, b_ref, o_ref, st_ref, acc):
    g = pl.program_id(0)
    r = x_ref.shape[0]
    ndf = nd_ref[0].astype(jnp.float32)
    xn = _normm(x_ref[...], st_in[...], ndf)
    o = _lrelu(jnp.dot(xn, w_ref[...], preferred_element_type=jnp.float32) + b_ref[...])
    o_ref[...] = o
    rows = jax.lax.broadcasted_iota(jnp.int32, (r, 1), 0) + g * r
    mask = (rows < nd_ref[0]).astype(jnp.float32)
    om = o * mask

    @pl.when(g == 0)
    def _():
        acc[...] = jnp.zeros_like(acc)

    acc[...] += jnp.stack([jnp.sum(om, axis=0), jnp.sum(o * om, axis=0)])
    st_ref[...] = acc[...]


def _q2(x, st, nd, w, b, r):
    n, cin = x.shape
    cout = w.shape[1]
    return pl.pallas_call(
        _q2_body,
        grid=(_row_grid(n, r),),
        in_specs=[_bspec(r, cin), _bspec(2, cin, True),
                  pl.BlockSpec(memory_space=pltpu.SMEM),
                  _bspec(cin, cout, True), _bspec(1, cout, True)],
        out_specs=[_bspec(r, cout), _bspec(2, cout, True)],
        out_shape=[jax.ShapeDtypeStruct((n, cout), jnp.float32),
                   jax.ShapeDtypeStruct((2, cout), jnp.float32)],
        scratch_shapes=[pltpu.VMEM((2, cout), jnp.float32)],
        compiler_params=pltpu.CompilerParams(dimension_semantics=("arbitrary",)),
    )(x, st, nd, w, b.reshape(1, -1))


def _q3_body(x_ref, st_in, nd_ref, w_ref, b_ref, wbot_ref, b2_ref):
    ndf = nd_ref[0].astype(jnp.float32)
    xn = _normm(x_ref[...], st_in[...], ndf)
    pp3 = _lrelu(jnp.dot(xn, w_ref[...], preferred_element_type=jnp.float32) + b_ref[...])
    b2_ref[...] = jnp.dot(pp3, wbot_ref[...], preferred_element_type=jnp.float32)


def _q3(x, st, nd, w, b, wbot, r):
    n, cin = x.shape
    return pl.pallas_call(
        _q3_body,
        grid=(_row_grid(n, r),),
        in_specs=[_bspec(r, cin), _bspec(2, cin, True),
                  pl.BlockSpec(memory_space=pltpu.SMEM),
                  _bspec(cin, C, True), _bspec(1, C, True), _bspec(C, C, True)],
        out_specs=[_bspec(r, C)],
        out_shape=[jax.ShapeDtypeStruct((n, C), jnp.float32)],
        compiler_params=pltpu.CompilerParams(dimension_semantics=("arbitrary",)),
    )(x, st, nd, w, b.reshape(1, -1), wbot)


def _final_body(acc_ref, cnt_ref, w_ref, b_ref, o_ref):
    a = jnp.concatenate([acc_ref[0, :HALF], acc_ref[1, 1:N_SCALE - HALF + 1]],
                        axis=0)
    cnt = jnp.concatenate([cnt_ref[0, :HALF, 0:1],
                           cnt_ref[1, 1:N_SCALE - HALF + 1, 0:1]], axis=0)
    m = a / jnp.clip(cnt, 1.0, None)
    o_ref[...] = jnp.dot(m, w_ref[...], preferred_element_type=jnp.float32) + b_ref[...]


def _final(acc2, cnt2, w, b):
    return pl.pallas_call(
        _final_body,
        in_specs=[pl.BlockSpec((2, ACC_R, C), lambda: (0, 0, 0)),
                  pl.BlockSpec((2, ACC_R, C), lambda: (0, 0, 0)),
                  pl.BlockSpec((C, C), lambda: (0, 0)),
                  pl.BlockSpec((1, C), lambda: (0, 0))],
        out_specs=[pl.BlockSpec((N_SCALE, C), lambda: (0, 0))],
        out_shape=[jax.ShapeDtypeStruct((N_SCALE, C), jnp.float32)],
    )(acc2, cnt2, w, b.reshape(1, -1))


HALF = 6272
ACC_R = 6400
_CH = 80
_NCHUNK = N_POINTS // _CH


def _sc_mesh():
    return plsc.VectorSubcoreMesh(core_axis_name="c", subcore_axis_name="s")


def _zero_fill(buf, rows, width):
    @pl.loop(0, rows)
    def zrow(r):
        zero16 = jnp.zeros((16,), jnp.float32)
        for j in range(width // 16):
            buf[r, pl.ds(j * 16, 16)] = zero16


def _seg_idx(cid, cis_v, idx_v):
    @pl.when(cid == 0)
    def _():
        for j in range(_CH // 16):
            v = cis_v[pl.ds(j * 16, 16)]
            idx_v[pl.ds(j * 16, 16)] = jnp.minimum(v, HALF)

    @pl.when(cid == 1)
    def _():
        for j in range(_CH // 16):
            v = cis_v[pl.ds(j * 16, 16)]
            idx_v[pl.ds(j * 16, 16)] = jnp.maximum(v - (HALF - 1), 0)


@functools.partial(
    pl.kernel,
    mesh=_sc_mesh(),
    out_type=jax.ShapeDtypeStruct((2, ACC_R, C), jnp.float32),
    scratch_types=[
        pltpu.VMEM((_CH,), jnp.int32),
        pltpu.VMEM((_CH,), jnp.int32),
        pltpu.VMEM((3, _CH), jnp.int32),
        pltpu.VMEM((_CH,), jnp.int32),
        pltpu.VMEM((_CH, C), jnp.float32),
        pltpu.VMEM((_CH, C), jnp.float32),
        pltpu.VMEM((_CH, C), jnp.float32),
        pltpu.VMEM((_CH, C), jnp.float32),
        pltpu.VMEM_SHARED((ACC_R, C), jnp.float32),
        pltpu.SemaphoreType.DMA,
        pltpu.SemaphoreType.DMA,
    ],
)
def _sc_main(a_hbm, b2_hbm, idx3_hbm, acc_out,
             cil_v, invc_v, idx3_v, idx_v, ra_v, rb_v, y_v, z_v, acc_sh,
             sem1, sem2):
    cid = lax.axis_index("c")
    sid = lax.axis_index("s")

    _zero_fill(z_v, _CH, C)

    @pl.loop(0, 5)
    def zcp(i):
        pltpu.sync_copy(z_v, acc_sh.at[pl.ds(sid * 400 + i * 80, 80)])

    plsc.subcore_barrier()

    @pl.loop(0, (_NCHUNK + 15) // 16)
    def chunk(i):
        k = sid + i * 16

        @pl.when(k < _NCHUNK)
        def _():
            base = k * _CH
            pltpu.sync_copy(idx3_hbm.at[k], idx3_v)
            for j in range(_CH // 16):
                cil_v[pl.ds(j * 16, 16)] = idx3_v[0, pl.ds(j * 16, 16)]
                invc_v[pl.ds(j * 16, 16)] = idx3_v[1, pl.ds(j * 16, 16)]
            cp_a = pltpu.async_copy(a_hbm.at[cil_v], ra_v, sem1)
            cp_b = pltpu.async_copy(b2_hbm.at[invc_v], rb_v, sem2)

            @pl.when(cid == 0)
            def _():
                for j in range(_CH // 16):
                    v = idx3_v[2, pl.ds(j * 16, 16)]
                    idx_v[pl.ds(j * 16, 16)] = jnp.minimum(v, HALF)

            @pl.when(cid == 1)
            def _():
                for j in range(_CH // 16):
                    v = idx3_v[2, pl.ds(j * 16, 16)]
                    idx_v[pl.ds(j * 16, 16)] = jnp.maximum(v - (HALF - 1), 0)

            cp_a.wait()
            cp_b.wait()

            @pl.loop(0, _CH)
            def row(r):
                for j in range(C // 16):
                    a = ra_v[r, pl.ds(j * 16, 16)] + rb_v[r, pl.ds(j * 16, 16)]
                    y_v[r, pl.ds(j * 16, 16)] = jnp.maximum(a, 0.1 * a)

            pltpu.sync_copy(y_v, acc_sh.at[idx_v], add=True)

    plsc.subcore_barrier()
    pltpu.sync_copy(acc_sh.at[pl.ds(sid * 400, 400)],
                    acc_out.at[cid, pl.ds(sid * 400, 400)])


@functools.partial(
    pl.kernel,
    mesh=_sc_mesh(),
    out_type=jax.ShapeDtypeStruct((2, ACC_R, C), jnp.float32),
    scratch_types=[
        pltpu.VMEM((_CH,), jnp.int32),
        pltpu.VMEM((_CH,), jnp.int32),
        pltpu.VMEM((_CH, C), jnp.float32),
        pltpu.VMEM((_CH, C), jnp.float32),
        pltpu.VMEM_SHARED((ACC_R, C), jnp.float32),
    ],
)
def _sc_cnt(cis_hbm, cnt_out, cis_v, idx_v, ones_v, z_v, acc_sh):
    cid = lax.axis_index("c")
    sid = lax.axis_index("s")

    _zero_fill(z_v, _CH, C)

    @pl.loop(0, 5)
    def zcp(i):
        pltpu.sync_copy(z_v, acc_sh.at[pl.ds(sid * 400 + i * 80, 80)])

    @pl.loop(0, _CH)
    def fill1(r):
        one16 = jnp.ones((16,), jnp.float32)
        for j in range(C // 16):
            ones_v[r, pl.ds(j * 16, 16)] = one16

    plsc.subcore_barrier()

    @pl.loop(0, (_NCHUNK + 15) // 16)
    def chunk(i):
        k = sid + i * 16

        @pl.when(k < _NCHUNK)
        def _():
            pltpu.sync_copy(cis_hbm.at[pl.ds(k * _CH, _CH)], cis_v)
            _seg_idx(cid, cis_v, idx_v)
            pltpu.sync_copy(ones_v, acc_sh.at[idx_v], add=True)

    plsc.subcore_barrier()
    pltpu.sync_copy(acc_sh.at[pl.ds(sid * 400, 400)],
                    cnt_out.at[cid, pl.ds(sid * 400, 400)])


@functools.partial(
    pl.kernel,
    mesh=_sc_mesh(),
    out_type=jax.ShapeDtypeStruct((N_POINTS, C), jnp.float32),
    scratch_types=[
        pltpu.VMEM((_CH,), jnp.int32),
        pltpu.VMEM((_CH, C), jnp.float32),
        pltpu.SemaphoreType.DMA,
    ],
)
def _sc_fgather(pfea_hbm, cis_hbm, out_hbm, cis_v, rows_v, sem):
    cid = lax.axis_index("c")
    sid = lax.axis_index("s")
    w = sid * 2 + cid

    @pl.loop(0, (_NCHUNK + 31) // 32)
    def chunk(i):
        k = w + i * 32

        @pl.when(k < _NCHUNK)
        def _():
            base = k * _CH
            pltpu.sync_copy(cis_hbm.at[pl.ds(base, _CH)], cis_v)
            pltpu.async_copy(pfea_hbm.at[cis_v], rows_v, sem).wait()
            pltpu.sync_copy(rows_v, out_hbm.at[pl.ds(base, _CH)])


def kernel(features, partial_features, params, coors, coors_inv_last, coors_inv_scale):
    p = params
    feat, A = _vchain(features, p, wpi=p['pi_W'], bpi=p['pi_b'],
                      wtop=p['po_W1'][:C], b1=p['po_b1'])
    loss = _vchain(partial_features, p, wlg=p['lg_W'], blg=p['lg_b'])[0, 0]

    key = ((coors[:, 0] << 18) + ((coors[:, 1] >> 1) << 12)
           + ((coors[:, 2] >> 1) << 6) + (coors[:, 3] >> 1))
    pres = jnp.zeros((2, KEYSPACE), jnp.int32).at[0, key].set(1)
    rank2d, nd2d = _rank(pres.reshape(2, 4096, C))
    inv = rank2d.reshape(-1)[key]
    nd = nd2d.reshape(-1)

    dsum = jnp.zeros((N_LAST, C), jnp.float32).at[inv].add(feat)
    dcnt = jnp.zeros((N_LAST, 1), jnp.float32).at[inv, 0].add(1.0)

    pp1, st1 = _q1(dsum, dcnt, nd, p['pp_W1'], p['pp_b1'], 1000)
    pp2, st2 = _q2(pp1, st1, nd, p['pp_W2'], p['pp_b2'], 1000)
    B2 = _q3(pp2, st2, nd, p['pp_W3'], p['pp_b3'], p['po_W1'][C:], 1000)[0]

    invc = inv[coors_inv_last]
    idx3 = jnp.stack([coors_inv_last, invc, coors_inv_scale]
                     ).reshape(3, _NCHUNK, _CH).transpose(1, 0, 2)
    acc2 = _sc_main(A, B2, idx3)
    cnt2 = _sc_cnt(coors_inv_scale)
    p_fea = _final(acc2, cnt2, p['po_W2'], p['po_b2'])[0]
    return (_sc_fgather(p_fea, coors_inv_scale), loss)
